# baseline (speedup 1.0000x reference)
import functools
import math

import jax
import jax.numpy as jnp
from jax import lax
from jax.experimental import pallas as pl
from jax.experimental.pallas import tpu as pltpu
from jax.experimental.pallas import tpu_sc as plsc

S = 200
B = 1024
D = 64
SCALE = 8.0

NC = 2
NS = 16
NW = NC * NS
BGRP = 4
SGRP = NW // BGRP
S_PER = S // SGRP
B_PER = B // BGRP
LANES = 16


def _make_pe(d_model, max_len):
    position = jnp.arange(0, max_len, dtype=jnp.float32)[:, None]
    div_term = jnp.exp(
        jnp.arange(0, d_model, 2, dtype=jnp.float32) * (-math.log(10000.0) / d_model)
    )
    pe = jnp.zeros((max_len, d_model), dtype=jnp.float32)
    pe = pe.at[:, 0::2].set(jnp.sin(position * div_term))
    pe = pe.at[:, 1::2].set(jnp.cos(position * div_term))
    return pe


@functools.partial(
    pl.kernel,
    mesh=plsc.VectorSubcoreMesh(core_axis_name="c", subcore_axis_name="s"),
    compiler_params=pltpu.CompilerParams(use_tc_tiling_on_sc=False),
    out_type=jax.ShapeDtypeStruct((S, B, D), jnp.float32),
    scratch_types=[
        pltpu.VMEM((B_PER,), jnp.int32),
        pltpu.VMEM((B_PER, D), jnp.float32),
        pltpu.VMEM((D,), jnp.float32),
        pltpu.SemaphoreType.DMA,
    ],
)
def _emb_kernel(xt_hbm, pe_hbm, table_hbm, out_hbm, idx_v, rows_v, pe_v, sem):
    wid = lax.axis_index("s") * NC + lax.axis_index("c")
    sgrp = wid // BGRP
    b0 = (wid % BGRP) * B_PER
    s_lo = sgrp * S_PER

    def body(i, carry):
        s = s_lo + i
        pltpu.sync_copy(xt_hbm.at[s, pl.ds(b0, B_PER)], idx_v)
        pltpu.async_copy(table_hbm.at[idx_v], rows_v, sem).wait()
        pltpu.sync_copy(pe_hbm.at[s], pe_v)
        pe_regs = [pe_v[pl.ds(LANES * c, LANES)] for c in range(D // LANES)]

        def row(r, rcarry):
            for c in range(D // LANES):
                sl = pl.ds(LANES * c, LANES)
                rows_v[r, sl] = rows_v[r, sl] * SCALE + pe_regs[c]
            return rcarry

        lax.fori_loop(0, B_PER, row, 0)
        pltpu.sync_copy(rows_v, out_hbm.at[s, pl.ds(b0, B_PER)])
        return carry

    lax.fori_loop(0, S_PER, body, 0)


def kernel(x, emb_table):
    xt = jnp.transpose(x.astype(jnp.int32), (1, 0))
    pe = _make_pe(D, S)
    return _emb_kernel(xt, pe, emb_table)
